# Initial kernel scaffold; baseline (speedup 1.0000x reference)
import jax
import jax.numpy as jnp
from jax.experimental import pallas as pl


def kernel(x, edge_index, params):
    raise NotImplementedError("write your pallas kernel here")


# speedup vs baseline: 31.3221x; 31.3221x over previous
import functools

import jax
import jax.numpy as jnp
from jax import lax
from jax.experimental import pallas as pl
from jax.experimental.pallas import tpu as pltpu
from jax.experimental.pallas import tpu_sc as plsc

F32 = jnp.float32

N_NODE = 10000
NP = 10240
N_EDGE = 640000
EP = 655360
DUMP = NP - 1
B = 4

NSC = 2
NTILE = 16
LANE = 16
NW = NSC * NTILE
EPW = EP // NW
CHUNK = 2048

EB = 5120


_SC_PARAMS = pltpu.CompilerParams(needs_layout_passes=False)


def _worker_id():
    return lax.axis_index("s") * NSC + lax.axis_index("c")


@functools.partial(jax.jit, static_argnames=("d", "nidx", "diff"))
def _sc_gather(table, idxs, *, d, nidx, diff):
    n_out = 1 if diff else nidx
    out_type = [jax.ShapeDtypeStruct((B, d, EP), F32) for _ in range(n_out)]
    scratch = (
        [pltpu.VMEM((d * NP,), F32)]
        + [pltpu.VMEM((CHUNK,), jnp.int32) for _ in range(nidx)]
        + [pltpu.VMEM((d, CHUNK), F32) for _ in range(n_out)]
    )
    mesh = plsc.VectorSubcoreMesh(core_axis_name="c", subcore_axis_name="s")

    def body(*refs):
        table_h = refs[0]
        idx_h = refs[1:1 + nidx]
        out_h = refs[1 + nidx:1 + nidx + n_out]
        sc = refs[1 + nidx + n_out:]
        tab_v = sc[0]
        idx_v = sc[1:1 + nidx]
        buf_v = sc[1 + nidx:]

        base = _worker_id() * EPW
        for b in range(B):
            pltpu.sync_copy(table_h.at[b], tab_v)
            for ck in range(EPW // CHUNK):
                for j in range(nidx):
                    pltpu.sync_copy(
                        idx_h[j].at[pl.ds(base + ck * CHUNK, CHUNK)],
                        idx_v[j])
                def inner(i, _, ck=ck):
                    gathered = []
                    for j in range(nidx):
                        iv = idx_v[j][pl.ds(i * LANE, LANE)]
                        gathered.append(
                            [plsc.load_gather(tab_v, [iv + dd * NP])
                             for dd in range(d)])
                    if diff:
                        for dd in range(d):
                            buf_v[0][dd, pl.ds(i * LANE, LANE)] = (
                                gathered[0][dd] - gathered[1][dd])
                    else:
                        for j in range(nidx):
                            for dd in range(d):
                                buf_v[j][dd, pl.ds(i * LANE, LANE)] = \
                                    gathered[j][dd]
                    return 0
                lax.fori_loop(0, CHUNK // LANE, inner, 0)
                for j in range(n_out):
                    pltpu.sync_copy(
                        buf_v[j],
                        out_h[j].at[b, :, pl.ds(base + ck * CHUNK, CHUNK)])

    k = pl.kernel(body, out_type=out_type, mesh=mesh, scratch_types=scratch,
                  compiler_params=_SC_PARAMS)
    res = k(table, *idxs)
    return res


@functools.partial(jax.jit, static_argnames=("d", "signs"))
def _sc_scatter(vals, idxs, *, d, signs):
    ns = len(signs)
    dn = d * NP
    out_type = jax.ShapeDtypeStruct((NW, B, dn), F32)
    scratch = (
        [pltpu.VMEM((dn,), F32)]
        + [pltpu.VMEM((CHUNK,), jnp.int32) for _ in range(ns)]
        + [pltpu.VMEM((d, CHUNK), F32) for _ in range(ns)]
    )
    mesh = plsc.VectorSubcoreMesh(core_axis_name="c", subcore_axis_name="s")

    def body(*refs):
        val_h = refs[0:ns]
        idx_h = refs[ns:2 * ns]
        out_h = refs[2 * ns]
        sc = refs[2 * ns + 1:]
        acc_v = sc[0]
        idx_v = sc[1:1 + ns]
        buf_v = sc[1 + ns:1 + 2 * ns]

        wid = _worker_id()
        base = wid * EPW
        zeros = jnp.zeros((LANE,), F32)
        for b in range(B):
            def zbody(i, _):
                acc_v[pl.ds(i * LANE, LANE)] = zeros
                return 0
            lax.fori_loop(0, dn // LANE, zbody, 0)
            for ck in range(EPW // CHUNK):
                for j in range(ns):
                    pltpu.sync_copy(
                        val_h[j].at[b, :, pl.ds(base + ck * CHUNK, CHUNK)],
                        buf_v[j])
                    pltpu.sync_copy(
                        idx_h[j].at[pl.ds(base + ck * CHUNK, CHUNK)],
                        idx_v[j])
                def inner(i, _, ck=ck):
                    for j in range(ns):
                        iv = idx_v[j][pl.ds(i * LANE, LANE)]
                        cnt, _last = plsc.scan_count(iv)
                        rounds = lax.reduce_max(cnt, axes=(0,))
                        vs = []
                        for dd in range(d):
                            v = buf_v[j][dd, pl.ds(i * LANE, LANE)]
                            if signs[j] < 0:
                                v = -v
                            vs.append(v)

                        def sbody(rr, _, iv=iv, cnt=cnt, vs=vs):
                            mk = cnt == rr
                            for dd in range(d):
                                plsc.addupdate_scatter(
                                    acc_v, [iv + dd * NP], vs[dd], mask=mk)
                            return 0
                        lax.fori_loop(0, rounds + 2, sbody, 0)
                    return 0
                lax.fori_loop(0, CHUNK // LANE, inner, 0)
            pltpu.sync_copy(acc_v, out_h.at[wid, b])

    k = pl.kernel(body, out_type=out_type, mesh=mesh, scratch_types=scratch,
                  compiler_params=_SC_PARAMS)
    return k(*vals, *idxs)


def _rb(v):
    return v.astype(jnp.bfloat16).astype(F32)


def _mlp_fwd(xs, w1, b1, w2, b2, din, dhid, dout):
    hs = []
    for j in range(dhid):
        if din == 1:
            acc = xs[0] * w1[0, j]
        else:
            acc = _rb(xs[0]) * _rb(w1[0, j])
            for dd in range(1, din):
                acc = acc + _rb(xs[dd]) * _rb(w1[dd, j])
        hs.append(jnp.tanh(acc + b1[j]))
    outs = []
    for o in range(dout):
        acc = _rb(hs[0]) * _rb(w2[0, o])
        for j in range(1, dhid):
            acc = acc + _rb(hs[j]) * _rb(w2[j, o])
        outs.append(acc + b2[o])
    return outs, hs


def _mlp_bwd(douts, hs, w1, w2, din, dhid, dout):
    dxs = [None] * din
    for j in range(dhid):
        if dout == 1:
            acc = douts[0] * w2[j, 0]
        else:
            acc = _rb(douts[0]) * _rb(w2[j, 0])
            for o in range(1, dout):
                acc = acc + _rb(douts[o]) * _rb(w2[j, o])
        dz = acc * (1.0 - hs[j] * hs[j])
        dzr = _rb(dz)
        for dd in range(din):
            t = dzr * _rb(w1[dd, j])
            dxs[dd] = t if dxs[dd] is None else dxs[dd] + t
    return dxs


def _smem_spec(shape):
    return pl.BlockSpec(shape, lambda *_: (0,) * len(shape),
                        memory_space=pltpu.SMEM)


def _edge_spec(d):
    return pl.BlockSpec((B, d, EB), lambda i: (0, 0, i))


def _wargs(p):
    return (p["W1"], p["b1"], p["W2"], p["b2"])


def _wspecs(p):
    return [_smem_spec(p["W1"].shape), _smem_spec(p["b1"].shape),
            _smem_spec(p["W2"].shape), _smem_spec(p["b2"].shape)]


def _edge_out(d):
    return jax.ShapeDtypeStruct((B, d, EP), F32)


_GRID = EP // EB


def _t1(e, p_f1, p_f2):
    def kern(e_ref, w11, b11, w21, b21, w12, b12, w22, b22,
             ee_ref, hf1_ref, nu_ref, hf2_ref):
        ev = e_ref[:, 0, :]
        ee, hf1 = _mlp_fwd([ev], w11, b11, w21, b21, 1, 5, 5)
        nu, hf2 = _mlp_fwd(ee, w12, b12, w22, b22, 5, 5, 5)
        for dd in range(5):
            ee_ref[:, dd, :] = ee[dd]
            hf1_ref[:, dd, :] = hf1[dd]
            nu_ref[:, dd, :] = nu[dd]
            hf2_ref[:, dd, :] = hf2[dd]

    return pl.pallas_call(
        kern,
        grid=(_GRID,),
        in_specs=[_edge_spec(1)] + _wspecs(p_f1) + _wspecs(p_f2),
        out_specs=[_edge_spec(5)] * 4,
        out_shape=[_edge_out(5)] * 4,
    )(e, *_wargs(p_f1), *_wargs(p_f2))


def _t2(edge_embed, nes, ner, p_f3, p_f4, p_g2):
    def kern(ee_ref, nes_ref, ner_ref,
             w13, b13, w23, b23, w14, b14, w24, b24, w1g, b1g, w2g, b2g,
             v2_ref, h3_ref, h4_ref, hg2_ref):
        m = [nes_ref[:, dd, :] * ner_ref[:, dd, :] for dd in range(5)]
        n11, h3 = _mlp_fwd(m, w13, b13, w23, b23, 5, 5, 5)
        n12, h4 = _mlp_fwd(m, w14, b14, w24, b24, 5, 5, 5)
        ee = [((ee_ref[:, dd, :] + n11[dd]) + (ee_ref[:, dd, :] + n12[dd]))
              * 0.5 for dd in range(5)]
        v2, hg2 = _mlp_fwd(ee, w1g, b1g, w2g, b2g, 5, 5, 1)
        v2_ref[:, 0, :] = v2[0]
        for dd in range(5):
            h3_ref[:, dd, :] = h3[dd]
            h4_ref[:, dd, :] = h4[dd]
            hg2_ref[:, dd, :] = hg2[dd]

    return pl.pallas_call(
        kern,
        grid=(_GRID,),
        in_specs=[_edge_spec(5)] * 3 + _wspecs(p_f3) + _wspecs(p_f4)
        + _wspecs(p_g2),
        out_specs=[_edge_spec(1)] + [_edge_spec(5)] * 3,
        out_shape=[_edge_out(1)] + [_edge_out(5)] * 3,
    )(edge_embed, nes, ner, *_wargs(p_f3), *_wargs(p_f4), *_wargs(p_g2))


def _t4(dvvs, hg2, h3, h4, nes, ner, p_f3, p_f4, p_g2):
    def kern(dvvs_ref, hg2_ref, h3_ref, h4_ref, nes_ref, ner_ref,
             w13, b13, w23, b23, w14, b14, w24, b24, w1g, b1g, w2g, b2g,
             dee_ref, dnes_ref, dner_ref):
        dv2 = dvvs_ref[:, 0, :]
        hg2 = [hg2_ref[:, dd, :] for dd in range(5)]
        dee = _mlp_bwd([dv2], hg2, w1g, w2g, 5, 5, 1)
        deeh = [0.5 * x for x in dee]
        h3 = [h3_ref[:, dd, :] for dd in range(5)]
        h4 = [h4_ref[:, dd, :] for dd in range(5)]
        dma = _mlp_bwd(deeh, h3, w13, w23, 5, 5, 5)
        dmb = _mlp_bwd(deeh, h4, w14, w24, 5, 5, 5)
        for dd in range(5):
            dm = dma[dd] + dmb[dd]
            dee_ref[:, dd, :] = dee[dd]
            dnes_ref[:, dd, :] = dm * ner_ref[:, dd, :]
            dner_ref[:, dd, :] = dm * nes_ref[:, dd, :]

    return pl.pallas_call(
        kern,
        grid=(_GRID,),
        in_specs=[_edge_spec(1)] + [_edge_spec(5)] * 5 + _wspecs(p_f3)
        + _wspecs(p_f4) + _wspecs(p_g2),
        out_specs=[_edge_spec(5)] * 3,
        out_shape=[_edge_out(5)] * 3,
    )(dvvs, hg2, h3, h4, nes, ner, *_wargs(p_f3), *_wargs(p_f4),
      *_wargs(p_g2))


def _t5(dnus, hf2, dee, hf1, p_f1, p_f2):
    def kern(dnus_ref, hf2_ref, dee_ref, hf1_ref,
             w11, b11, w21, b21, w12, b12, w22, b22, de_ref):
        dnu = [dnus_ref[:, dd, :] for dd in range(5)]
        hf2 = [hf2_ref[:, dd, :] for dd in range(5)]
        dee2 = _mlp_bwd(dnu, hf2, w12, w22, 5, 5, 5)
        deet = [dee_ref[:, dd, :] + dee2[dd] for dd in range(5)]
        hf1 = [hf1_ref[:, dd, :] for dd in range(5)]
        de = _mlp_bwd(deet, hf1, w11, w21, 1, 5, 5)
        de_ref[:, 0, :] = de[0]

    return pl.pallas_call(
        kern,
        grid=(_GRID,),
        in_specs=[_edge_spec(5)] * 4 + _wspecs(p_f1) + _wspecs(p_f2),
        out_specs=_edge_spec(1),
        out_shape=_edge_out(1),
    )(dnus, hf2, dee, hf1, *_wargs(p_f1), *_wargs(p_f2))


def _fold(parts, extra=None, neg=False):
    nw, b_, w = parts.shape
    wb = 2560
    grid = w // wb

    def kern(*refs):
        if extra is None:
            p_ref, o_ref = refs
            s = jnp.sum(p_ref[...], axis=0)
        else:
            p_ref, e_ref, o_ref = refs
            s = jnp.sum(p_ref[...], axis=0) + e_ref[...]
        o_ref[...] = -s if neg else s

    in_specs = [pl.BlockSpec((nw, B, wb), lambda i: (0, 0, i))]
    args = [parts]
    if extra is not None:
        in_specs.append(pl.BlockSpec((B, wb), lambda i: (0, i)))
        args.append(extra)
    return pl.pallas_call(
        kern,
        grid=(grid,),
        in_specs=in_specs,
        out_specs=pl.BlockSpec((B, wb), lambda i: (0, i)),
        out_shape=jax.ShapeDtypeStruct((B, w), F32),
    )(*args)


def _t3(qpad, ppad, ne, vv_in, params):
    p_v1, p_v3, p_v, p_t = (params["V1"], params["V3"], params["V"],
                            params["T"])
    w1n2 = jnp.pad(params["net2"]["W1"], ((0, NP - N_NODE), (0, 0)))
    b1n2 = params["net2"]["b1"].reshape(1, -1)
    w2n2 = params["net2"]["W2"].reshape(1, -1)
    h2 = w1n2.shape[1]

    def kern(q_ref, p_ref, ne_ref, vv_ref, w1n2_ref, b1n2_ref, w2n2_ref,
             w1v1, b1v1, w2v1, b2v1, w1v3, b1v3, w2v3, b2v3,
             w1v, b1v, w2v, b2v, w1t, b1t, w2t, b2t,
             dvv_ref, dne_ref, dqn_ref, dt_ref):
        q = q_ref[...]
        vv = vv_ref[...]
        ne = [ne_ref[:, dd, :] for dd in range(5)]
        v1, hg1 = _mlp_fwd(ne, w1v1, b1v1, w2v1, b2v1, 5, 5, 1)
        v3, hg3 = _mlp_fwd([q], w1v3, b1v3, w2v3, b2v3, 1, 5, 1)
        u = [v1[0], vv, v3[0]]
        vth, hgv = _mlp_fwd(u, w1v, b1v, w2v, b2v, 3, 5, 1)
        h11 = vth[0]
        z = jnp.dot(h11, w1n2_ref[...],
                    preferred_element_type=F32) + b1n2_ref[...]
        a = jnp.tanh(z)
        g = (1.0 - a * a) * w2n2_ref[...]
        dh11 = lax.dot_general(g, w1n2_ref[...],
                               (((1,), (1,)), ((), ())),
                               preferred_element_type=F32)
        du = _mlp_bwd([dh11], hgv, w1v, w2v, 3, 5, 1)
        dvv_ref[...] = du[1]
        dqn_ref[...] = _mlp_bwd([du[2]], hg3, w1v3, w2v3, 1, 5, 1)[0]
        dne = _mlp_bwd([du[0]], hg1, w1v1, w2v1, 5, 5, 1)
        for dd in range(5):
            dne_ref[:, dd, :] = dne[dd]
        pv = p_ref[...]
        dt = None
        for j in range(10):
            ht = jnp.tanh(pv * w1t[0, j] + b1t[j])
            dz = w2t[j, 0] * (1.0 - ht * ht)
            t = _rb(dz) * _rb(w1t[0, j])
            dt = t if dt is None else dt + t
        dt_ref[...] = dt

    vm = lambda shape: pl.BlockSpec(shape, lambda: (0,) * len(shape))
    return pl.pallas_call(
        kern,
        in_specs=[vm((B, NP)), vm((B, NP)), vm((B, 5, NP)),
                  vm((B, NP)), vm((NP, h2)), vm((1, h2)), vm((1, h2))]
        + _wspecs(p_v1) + _wspecs(p_v3) + _wspecs(p_v) + _wspecs(p_t),
        out_specs=[vm((B, NP)), vm((B, 5, NP)), vm((B, NP)), vm((B, NP))],
        out_shape=[jax.ShapeDtypeStruct((B, NP), F32),
                   jax.ShapeDtypeStruct((B, 5, NP), F32),
                   jax.ShapeDtypeStruct((B, NP), F32),
                   jax.ShapeDtypeStruct((B, NP), F32)],
    )(qpad, ppad, ne, vv_in, w1n2, b1n2, w2n2,
      *_wargs(p_v1), *_wargs(p_v3), *_wargs(p_v), *_wargs(p_t))


def kernel(x, edge_index, params):
    n = N_NODE
    q = x[:, 0:n]
    p = x[:, n:2 * n]
    s = jnp.pad(edge_index[0], (0, EP - N_EDGE), constant_values=DUMP)
    r = jnp.pad(edge_index[1], (0, EP - N_EDGE), constant_values=DUMP)
    qpad = jnp.pad(q, ((0, 0), (0, NP - n)))
    ppad = jnp.pad(p, ((0, 0), (0, NP - n)))

    e = _sc_gather(qpad, (s, r), d=1, nidx=2, diff=True)[0]
    edge_embed, hf1, nu, hf2 = _t1(e, params["edge_init"],
                                   params["node_update"])
    nep = _sc_scatter((nu,), (s,), d=5, signs=(1,))
    ne_flat = _fold(nep)
    ne = ne_flat.reshape(B, 5, NP)

    nes, ner = _sc_gather(ne_flat, (s, r), d=5, nidx=2, diff=False)
    v2, h3, h4, hg2 = _t2(edge_embed, nes, ner, params["edge_up1"],
                          params["edge_up2"], params["V2"])
    vvp = _sc_scatter((v2,), (s,), d=1, signs=(1,))
    vv = _fold(vvp)

    dvv, dne_node, dqn, dt = _t3(qpad, ppad, ne, vv, params)

    dvvs = _sc_gather(dvv, (s,), d=1, nidx=1, diff=False)[0]
    dee, dnes, dner = _t4(dvvs, hg2, h3, h4, nes, ner,
                          params["edge_up1"], params["edge_up2"],
                          params["V2"])
    dnep = _sc_scatter((dnes, dner), (s, r), d=5, signs=(1, 1))
    dne_flat = _fold(dnep, extra=dne_node.reshape(B, 5 * NP))

    dnus = _sc_gather(dne_flat, (s,), d=5, nidx=1, diff=False)[0]
    de = _t5(dnus, hf2, dee, hf1, params["edge_init"],
             params["node_update"])
    dqp = _sc_scatter((de, de), (s, r), d=1, signs=(1, -1))
    mdq = _fold(dqp, extra=dqn, neg=True)

    return jnp.concatenate([dt[:, :n], mdq[:, :n]], axis=1)


# speedup vs baseline: 31.4541x; 1.0042x over previous
import functools

import jax
import jax.numpy as jnp
from jax import lax
from jax.experimental import pallas as pl
from jax.experimental.pallas import tpu as pltpu
from jax.experimental.pallas import tpu_sc as plsc

F32 = jnp.float32

N_NODE = 10000
NP = 10240
N_EDGE = 640000
EP = 655360
DUMP = NP - 1
B = 4

NSC = 2
NTILE = 16
LANE = 16
NW = NSC * NTILE
EPW = EP // NW
CHUNK = 2048

EB = 5120


_SC_PARAMS = pltpu.CompilerParams(needs_layout_passes=False)


def _worker_id():
    return lax.axis_index("s") * NSC + lax.axis_index("c")


@functools.partial(jax.jit, static_argnames=("d", "nidx", "diff"))
def _sc_gather(table, idxs, *, d, nidx, diff):
    n_out = 1 if diff else nidx
    out_type = [jax.ShapeDtypeStruct((B, d, EP), F32) for _ in range(n_out)]
    scratch = (
        [pltpu.VMEM((d * NP,), F32)]
        + [pltpu.VMEM((CHUNK,), jnp.int32) for _ in range(nidx)]
        + [pltpu.VMEM((d, CHUNK), F32) for _ in range(n_out)]
    )
    mesh = plsc.VectorSubcoreMesh(core_axis_name="c", subcore_axis_name="s")

    def body(*refs):
        table_h = refs[0]
        idx_h = refs[1:1 + nidx]
        out_h = refs[1 + nidx:1 + nidx + n_out]
        sc = refs[1 + nidx + n_out:]
        tab_v = sc[0]
        idx_v = sc[1:1 + nidx]
        buf_v = sc[1 + nidx:]

        base = _worker_id() * EPW
        for b in range(B):
            pltpu.sync_copy(table_h.at[b], tab_v)
            for ck in range(EPW // CHUNK):
                for j in range(nidx):
                    pltpu.sync_copy(
                        idx_h[j].at[pl.ds(base + ck * CHUNK, CHUNK)],
                        idx_v[j])
                def inner(i, _, ck=ck):
                    gathered = []
                    for j in range(nidx):
                        iv = idx_v[j][pl.ds(i * LANE, LANE)]
                        gathered.append(
                            [plsc.load_gather(tab_v, [iv + dd * NP])
                             for dd in range(d)])
                    if diff:
                        for dd in range(d):
                            buf_v[0][dd, pl.ds(i * LANE, LANE)] = (
                                gathered[0][dd] - gathered[1][dd])
                    else:
                        for j in range(nidx):
                            for dd in range(d):
                                buf_v[j][dd, pl.ds(i * LANE, LANE)] = \
                                    gathered[j][dd]
                    return 0
                lax.fori_loop(0, CHUNK // LANE, inner, 0)
                for j in range(n_out):
                    pltpu.sync_copy(
                        buf_v[j],
                        out_h[j].at[b, :, pl.ds(base + ck * CHUNK, CHUNK)])

    k = pl.kernel(body, out_type=out_type, mesh=mesh, scratch_types=scratch,
                  compiler_params=_SC_PARAMS)
    res = k(table, *idxs)
    return res


@functools.partial(jax.jit, static_argnames=("d", "signs"))
def _sc_scatter(vals, idxs, *, d, signs):
    ns = len(signs)
    dn = d * NP
    out_type = jax.ShapeDtypeStruct((NW, B, dn), F32)
    scratch = (
        [pltpu.VMEM((dn,), F32)]
        + [pltpu.VMEM((CHUNK,), jnp.int32) for _ in range(ns)]
        + [pltpu.VMEM((d, CHUNK), F32) for _ in range(ns)]
    )
    mesh = plsc.VectorSubcoreMesh(core_axis_name="c", subcore_axis_name="s")

    def body(*refs):
        val_h = refs[0:ns]
        idx_h = refs[ns:2 * ns]
        out_h = refs[2 * ns]
        sc = refs[2 * ns + 1:]
        acc_v = sc[0]
        idx_v = sc[1:1 + ns]
        buf_v = sc[1 + ns:1 + 2 * ns]

        wid = _worker_id()
        base = wid * EPW
        zeros = jnp.zeros((LANE,), F32)
        for b in range(B):
            def zbody(i, _):
                for u in range(8):
                    acc_v[pl.ds(i * 8 * LANE + u * LANE, LANE)] = zeros
                return 0
            lax.fori_loop(0, dn // (8 * LANE), zbody, 0)
            for ck in range(EPW // CHUNK):
                for j in range(ns):
                    pltpu.sync_copy(
                        val_h[j].at[b, :, pl.ds(base + ck * CHUNK, CHUNK)],
                        buf_v[j])
                    pltpu.sync_copy(
                        idx_h[j].at[pl.ds(base + ck * CHUNK, CHUNK)],
                        idx_v[j])
                def inner(i, _, ck=ck):
                    for j in range(ns):
                        iv = idx_v[j][pl.ds(i * LANE, LANE)]
                        cnt, _last = plsc.scan_count(iv)
                        vs = []
                        for dd in range(d):
                            v = buf_v[j][dd, pl.ds(i * LANE, LANE)]
                            if signs[j] < 0:
                                v = -v
                            vs.append(v)
                        mk1 = cnt == 1
                        for dd in range(d):
                            plsc.addupdate_scatter(
                                acc_v, [iv + dd * NP], vs[dd], mask=mk1)
                        dups = plsc.all_reduce_population_count(cnt > 1)

                        @pl.when(dups[0] > 0)
                        def _(iv=iv, cnt=cnt, vs=vs):
                            rounds = lax.reduce_max(cnt, axes=(0,))

                            def sbody(rr, _, iv=iv, cnt=cnt, vs=vs):
                                mk = cnt == rr
                                for dd in range(d):
                                    plsc.addupdate_scatter(
                                        acc_v, [iv + dd * NP], vs[dd],
                                        mask=mk)
                                return 0
                            lax.fori_loop(2, rounds + 1, sbody, 0)
                    return 0
                lax.fori_loop(0, CHUNK // LANE, inner, 0)
            pltpu.sync_copy(acc_v, out_h.at[wid, b])

    k = pl.kernel(body, out_type=out_type, mesh=mesh, scratch_types=scratch,
                  compiler_params=_SC_PARAMS)
    return k(*vals, *idxs)


def _rb(v):
    return v.astype(jnp.bfloat16).astype(F32)


def _mlp_fwd(xs, w1, b1, w2, b2, din, dhid, dout):
    hs = []
    for j in range(dhid):
        if din == 1:
            acc = xs[0] * w1[0, j]
        else:
            acc = _rb(xs[0]) * _rb(w1[0, j])
            for dd in range(1, din):
                acc = acc + _rb(xs[dd]) * _rb(w1[dd, j])
        hs.append(jnp.tanh(acc + b1[j]))
    outs = []
    for o in range(dout):
        acc = _rb(hs[0]) * _rb(w2[0, o])
        for j in range(1, dhid):
            acc = acc + _rb(hs[j]) * _rb(w2[j, o])
        outs.append(acc + b2[o])
    return outs, hs


def _mlp_bwd(douts, hs, w1, w2, din, dhid, dout):
    dxs = [None] * din
    for j in range(dhid):
        if dout == 1:
            acc = douts[0] * w2[j, 0]
        else:
            acc = _rb(douts[0]) * _rb(w2[j, 0])
            for o in range(1, dout):
                acc = acc + _rb(douts[o]) * _rb(w2[j, o])
        dz = acc * (1.0 - hs[j] * hs[j])
        dzr = _rb(dz)
        for dd in range(din):
            t = dzr * _rb(w1[dd, j])
            dxs[dd] = t if dxs[dd] is None else dxs[dd] + t
    return dxs


def _smem_spec(shape):
    return pl.BlockSpec(shape, lambda *_: (0,) * len(shape),
                        memory_space=pltpu.SMEM)


def _edge_spec(d):
    return pl.BlockSpec((B, d, EB), lambda i: (0, 0, i))


def _wargs(p):
    return (p["W1"], p["b1"], p["W2"], p["b2"])


def _wspecs(p):
    return [_smem_spec(p["W1"].shape), _smem_spec(p["b1"].shape),
            _smem_spec(p["W2"].shape), _smem_spec(p["b2"].shape)]


def _edge_out(d):
    return jax.ShapeDtypeStruct((B, d, EP), F32)


_GRID = EP // EB


def _t1(e, p_f1, p_f2):
    def kern(e_ref, w11, b11, w21, b21, w12, b12, w22, b22,
             ee_ref, hf1_ref, nu_ref, hf2_ref):
        ev = e_ref[:, 0, :]
        ee, hf1 = _mlp_fwd([ev], w11, b11, w21, b21, 1, 5, 5)
        nu, hf2 = _mlp_fwd(ee, w12, b12, w22, b22, 5, 5, 5)
        for dd in range(5):
            ee_ref[:, dd, :] = ee[dd]
            hf1_ref[:, dd, :] = hf1[dd]
            nu_ref[:, dd, :] = nu[dd]
            hf2_ref[:, dd, :] = hf2[dd]

    return pl.pallas_call(
        kern,
        grid=(_GRID,),
        in_specs=[_edge_spec(1)] + _wspecs(p_f1) + _wspecs(p_f2),
        out_specs=[_edge_spec(5)] * 4,
        out_shape=[_edge_out(5)] * 4,
    )(e, *_wargs(p_f1), *_wargs(p_f2))


def _t2(edge_embed, nes, ner, p_f3, p_f4, p_g2):
    def kern(ee_ref, nes_ref, ner_ref,
             w13, b13, w23, b23, w14, b14, w24, b24, w1g, b1g, w2g, b2g,
             v2_ref, h3_ref, h4_ref, hg2_ref):
        m = [nes_ref[:, dd, :] * ner_ref[:, dd, :] for dd in range(5)]
        n11, h3 = _mlp_fwd(m, w13, b13, w23, b23, 5, 5, 5)
        n12, h4 = _mlp_fwd(m, w14, b14, w24, b24, 5, 5, 5)
        ee = [((ee_ref[:, dd, :] + n11[dd]) + (ee_ref[:, dd, :] + n12[dd]))
              * 0.5 for dd in range(5)]
        v2, hg2 = _mlp_fwd(ee, w1g, b1g, w2g, b2g, 5, 5, 1)
        v2_ref[:, 0, :] = v2[0]
        for dd in range(5):
            h3_ref[:, dd, :] = h3[dd]
            h4_ref[:, dd, :] = h4[dd]
            hg2_ref[:, dd, :] = hg2[dd]

    return pl.pallas_call(
        kern,
        grid=(_GRID,),
        in_specs=[_edge_spec(5)] * 3 + _wspecs(p_f3) + _wspecs(p_f4)
        + _wspecs(p_g2),
        out_specs=[_edge_spec(1)] + [_edge_spec(5)] * 3,
        out_shape=[_edge_out(1)] + [_edge_out(5)] * 3,
    )(edge_embed, nes, ner, *_wargs(p_f3), *_wargs(p_f4), *_wargs(p_g2))


def _t4(dvvs, hg2, h3, h4, nes, ner, p_f3, p_f4, p_g2):
    def kern(dvvs_ref, hg2_ref, h3_ref, h4_ref, nes_ref, ner_ref,
             w13, b13, w23, b23, w14, b14, w24, b24, w1g, b1g, w2g, b2g,
             dee_ref, dnes_ref, dner_ref):
        dv2 = dvvs_ref[:, 0, :]
        hg2 = [hg2_ref[:, dd, :] for dd in range(5)]
        dee = _mlp_bwd([dv2], hg2, w1g, w2g, 5, 5, 1)
        deeh = [0.5 * x for x in dee]
        h3 = [h3_ref[:, dd, :] for dd in range(5)]
        h4 = [h4_ref[:, dd, :] for dd in range(5)]
        dma = _mlp_bwd(deeh, h3, w13, w23, 5, 5, 5)
        dmb = _mlp_bwd(deeh, h4, w14, w24, 5, 5, 5)
        for dd in range(5):
            dm = dma[dd] + dmb[dd]
            dee_ref[:, dd, :] = dee[dd]
            dnes_ref[:, dd, :] = dm * ner_ref[:, dd, :]
            dner_ref[:, dd, :] = dm * nes_ref[:, dd, :]

    return pl.pallas_call(
        kern,
        grid=(_GRID,),
        in_specs=[_edge_spec(1)] + [_edge_spec(5)] * 5 + _wspecs(p_f3)
        + _wspecs(p_f4) + _wspecs(p_g2),
        out_specs=[_edge_spec(5)] * 3,
        out_shape=[_edge_out(5)] * 3,
    )(dvvs, hg2, h3, h4, nes, ner, *_wargs(p_f3), *_wargs(p_f4),
      *_wargs(p_g2))


def _t5(dnus, hf2, dee, hf1, p_f1, p_f2):
    def kern(dnus_ref, hf2_ref, dee_ref, hf1_ref,
             w11, b11, w21, b21, w12, b12, w22, b22, de_ref):
        dnu = [dnus_ref[:, dd, :] for dd in range(5)]
        hf2 = [hf2_ref[:, dd, :] for dd in range(5)]
        dee2 = _mlp_bwd(dnu, hf2, w12, w22, 5, 5, 5)
        deet = [dee_ref[:, dd, :] + dee2[dd] for dd in range(5)]
        hf1 = [hf1_ref[:, dd, :] for dd in range(5)]
        de = _mlp_bwd(deet, hf1, w11, w21, 1, 5, 5)
        de_ref[:, 0, :] = de[0]

    return pl.pallas_call(
        kern,
        grid=(_GRID,),
        in_specs=[_edge_spec(5)] * 4 + _wspecs(p_f1) + _wspecs(p_f2),
        out_specs=_edge_spec(1),
        out_shape=_edge_out(1),
    )(dnus, hf2, dee, hf1, *_wargs(p_f1), *_wargs(p_f2))


def _fold(parts, extra=None, neg=False):
    nw, b_, w = parts.shape
    wb = 2560
    grid = w // wb

    def kern(*refs):
        if extra is None:
            p_ref, o_ref = refs
            s = jnp.sum(p_ref[...], axis=0)
        else:
            p_ref, e_ref, o_ref = refs
            s = jnp.sum(p_ref[...], axis=0) + e_ref[...]
        o_ref[...] = -s if neg else s

    in_specs = [pl.BlockSpec((nw, B, wb), lambda i: (0, 0, i))]
    args = [parts]
    if extra is not None:
        in_specs.append(pl.BlockSpec((B, wb), lambda i: (0, i)))
        args.append(extra)
    return pl.pallas_call(
        kern,
        grid=(grid,),
        in_specs=in_specs,
        out_specs=pl.BlockSpec((B, wb), lambda i: (0, i)),
        out_shape=jax.ShapeDtypeStruct((B, w), F32),
    )(*args)


def _t3(qpad, ppad, ne, vv_in, params):
    p_v1, p_v3, p_v, p_t = (params["V1"], params["V3"], params["V"],
                            params["T"])
    w1n2 = jnp.pad(params["net2"]["W1"], ((0, NP - N_NODE), (0, 0)))
    b1n2 = params["net2"]["b1"].reshape(1, -1)
    w2n2 = params["net2"]["W2"].reshape(1, -1)
    h2 = w1n2.shape[1]

    def kern(q_ref, p_ref, ne_ref, vv_ref, w1n2_ref, b1n2_ref, w2n2_ref,
             w1v1, b1v1, w2v1, b2v1, w1v3, b1v3, w2v3, b2v3,
             w1v, b1v, w2v, b2v, w1t, b1t, w2t, b2t,
             dvv_ref, dne_ref, dqn_ref, dt_ref):
        q = q_ref[...]
        vv = vv_ref[...]
        ne = [ne_ref[:, dd, :] for dd in range(5)]
        v1, hg1 = _mlp_fwd(ne, w1v1, b1v1, w2v1, b2v1, 5, 5, 1)
        v3, hg3 = _mlp_fwd([q], w1v3, b1v3, w2v3, b2v3, 1, 5, 1)
        u = [v1[0], vv, v3[0]]
        vth, hgv = _mlp_fwd(u, w1v, b1v, w2v, b2v, 3, 5, 1)
        h11 = vth[0]
        z = jnp.dot(h11, w1n2_ref[...],
                    preferred_element_type=F32) + b1n2_ref[...]
        a = jnp.tanh(z)
        g = (1.0 - a * a) * w2n2_ref[...]
        dh11 = lax.dot_general(g, w1n2_ref[...],
                               (((1,), (1,)), ((), ())),
                               preferred_element_type=F32)
        du = _mlp_bwd([dh11], hgv, w1v, w2v, 3, 5, 1)
        dvv_ref[...] = du[1]
        dqn_ref[...] = _mlp_bwd([du[2]], hg3, w1v3, w2v3, 1, 5, 1)[0]
        dne = _mlp_bwd([du[0]], hg1, w1v1, w2v1, 5, 5, 1)
        for dd in range(5):
            dne_ref[:, dd, :] = dne[dd]
        pv = p_ref[...]
        dt = None
        for j in range(10):
            ht = jnp.tanh(pv * w1t[0, j] + b1t[j])
            dz = w2t[j, 0] * (1.0 - ht * ht)
            t = _rb(dz) * _rb(w1t[0, j])
            dt = t if dt is None else dt + t
        dt_ref[...] = dt

    vm = lambda shape: pl.BlockSpec(shape, lambda: (0,) * len(shape))
    return pl.pallas_call(
        kern,
        in_specs=[vm((B, NP)), vm((B, NP)), vm((B, 5, NP)),
                  vm((B, NP)), vm((NP, h2)), vm((1, h2)), vm((1, h2))]
        + _wspecs(p_v1) + _wspecs(p_v3) + _wspecs(p_v) + _wspecs(p_t),
        out_specs=[vm((B, NP)), vm((B, 5, NP)), vm((B, NP)), vm((B, NP))],
        out_shape=[jax.ShapeDtypeStruct((B, NP), F32),
                   jax.ShapeDtypeStruct((B, 5, NP), F32),
                   jax.ShapeDtypeStruct((B, NP), F32),
                   jax.ShapeDtypeStruct((B, NP), F32)],
    )(qpad, ppad, ne, vv_in, w1n2, b1n2, w2n2,
      *_wargs(p_v1), *_wargs(p_v3), *_wargs(p_v), *_wargs(p_t))


def kernel(x, edge_index, params):
    n = N_NODE
    q = x[:, 0:n]
    p = x[:, n:2 * n]
    s = jnp.pad(edge_index[0], (0, EP - N_EDGE), constant_values=DUMP)
    r = jnp.pad(edge_index[1], (0, EP - N_EDGE), constant_values=DUMP)
    qpad = jnp.pad(q, ((0, 0), (0, NP - n)))
    ppad = jnp.pad(p, ((0, 0), (0, NP - n)))

    e = _sc_gather(qpad, (s, r), d=1, nidx=2, diff=True)[0]
    edge_embed, hf1, nu, hf2 = _t1(e, params["edge_init"],
                                   params["node_update"])
    nep = _sc_scatter((nu,), (s,), d=5, signs=(1,))
    ne_flat = _fold(nep)
    ne = ne_flat.reshape(B, 5, NP)

    nes, ner = _sc_gather(ne_flat, (s, r), d=5, nidx=2, diff=False)
    v2, h3, h4, hg2 = _t2(edge_embed, nes, ner, params["edge_up1"],
                          params["edge_up2"], params["V2"])
    vvp = _sc_scatter((v2,), (s,), d=1, signs=(1,))
    vv = _fold(vvp)

    dvv, dne_node, dqn, dt = _t3(qpad, ppad, ne, vv, params)

    dvvs = _sc_gather(dvv, (s,), d=1, nidx=1, diff=False)[0]
    dee, dnes, dner = _t4(dvvs, hg2, h3, h4, nes, ner,
                          params["edge_up1"], params["edge_up2"],
                          params["V2"])
    dnep = _sc_scatter((dnes, dner), (s, r), d=5, signs=(1, 1))
    dne_flat = _fold(dnep, extra=dne_node.reshape(B, 5 * NP))

    dnus = _sc_gather(dne_flat, (s,), d=5, nidx=1, diff=False)[0]
    de = _t5(dnus, hf2, dee, hf1, params["edge_init"],
             params["node_update"])
    dqp = _sc_scatter((de, de), (s, r), d=1, signs=(1, -1))
    mdq = _fold(dqp, extra=dqn, neg=True)

    return jnp.concatenate([dt[:, :n], mdq[:, :n]], axis=1)


# speedup vs baseline: 31.9694x; 1.0164x over previous
import functools

import jax
import jax.numpy as jnp
from jax import lax
from jax.experimental import pallas as pl
from jax.experimental.pallas import tpu as pltpu
from jax.experimental.pallas import tpu_sc as plsc

F32 = jnp.float32

N_NODE = 10000
NP = 10240
N_EDGE = 640000
EP = 655360
DUMP = NP - 1
B = 4

NSC = 2
NTILE = 16
LANE = 16
NW = NSC * NTILE
EPW = EP // NW
CHUNK = 2048

EB = 5120


_SC_PARAMS = pltpu.CompilerParams(needs_layout_passes=False)


def _worker_id():
    return lax.axis_index("s") * NSC + lax.axis_index("c")


@functools.partial(jax.jit, static_argnames=("d", "nidx", "diff"))
def _sc_gather(table, idxs, *, d, nidx, diff):
    n_out = 1 if diff else nidx
    out_type = [jax.ShapeDtypeStruct((B, d, EP), F32) for _ in range(n_out)]
    scratch = (
        [pltpu.VMEM((d * NP,), F32)]
        + [pltpu.VMEM((CHUNK,), jnp.int32) for _ in range(nidx)]
        + [pltpu.VMEM((d, CHUNK), F32) for _ in range(n_out)]
    )
    mesh = plsc.VectorSubcoreMesh(core_axis_name="c", subcore_axis_name="s")

    def body(*refs):
        table_h = refs[0]
        idx_h = refs[1:1 + nidx]
        out_h = refs[1 + nidx:1 + nidx + n_out]
        sc = refs[1 + nidx + n_out:]
        tab_v = sc[0]
        idx_v = sc[1:1 + nidx]
        buf_v = sc[1 + nidx:]

        base = _worker_id() * EPW
        for b in range(B):
            pltpu.sync_copy(table_h.at[b], tab_v)
            for ck in range(EPW // CHUNK):
                for j in range(nidx):
                    pltpu.sync_copy(
                        idx_h[j].at[pl.ds(base + ck * CHUNK, CHUNK)],
                        idx_v[j])
                def inner(i, _, ck=ck):
                    gathered = []
                    for j in range(nidx):
                        iv = idx_v[j][pl.ds(i * LANE, LANE)]
                        gathered.append(
                            [plsc.load_gather(tab_v, [iv + dd * NP])
                             for dd in range(d)])
                    if diff:
                        for dd in range(d):
                            buf_v[0][dd, pl.ds(i * LANE, LANE)] = (
                                gathered[0][dd] - gathered[1][dd])
                    else:
                        for j in range(nidx):
                            for dd in range(d):
                                buf_v[j][dd, pl.ds(i * LANE, LANE)] = \
                                    gathered[j][dd]
                    return 0
                lax.fori_loop(0, CHUNK // LANE, inner, 0)
                for j in range(n_out):
                    pltpu.sync_copy(
                        buf_v[j],
                        out_h[j].at[b, :, pl.ds(base + ck * CHUNK, CHUNK)])

    k = pl.kernel(body, out_type=out_type, mesh=mesh, scratch_types=scratch,
                  compiler_params=_SC_PARAMS)
    res = k(table, *idxs)
    return res


@jax.jit
def _sc_count(idxs):
    nidx = len(idxs)
    out_type = [jax.ShapeDtypeStruct((EP,), jnp.int32) for _ in range(nidx)]
    scratch = (
        [pltpu.VMEM((CHUNK,), jnp.int32) for _ in range(nidx)]
        + [pltpu.VMEM((CHUNK,), jnp.int32) for _ in range(nidx)]
    )
    mesh = plsc.VectorSubcoreMesh(core_axis_name="c", subcore_axis_name="s")

    def body(*refs):
        idx_h = refs[0:nidx]
        out_h = refs[nidx:2 * nidx]
        idx_v = refs[2 * nidx:3 * nidx]
        cnt_v = refs[3 * nidx:]

        base = _worker_id() * EPW
        for ck in range(EPW // CHUNK):
            for j in range(nidx):
                pltpu.sync_copy(
                    idx_h[j].at[pl.ds(base + ck * CHUNK, CHUNK)], idx_v[j])

                def inner(i, _, j=j):
                    iv = idx_v[j][pl.ds(i * LANE, LANE)]
                    cnt, _last = plsc.scan_count(iv)
                    cnt_v[j][pl.ds(i * LANE, LANE)] = cnt
                    return 0
                lax.fori_loop(0, CHUNK // LANE, inner, 0)
                pltpu.sync_copy(
                    cnt_v[j], out_h[j].at[pl.ds(base + ck * CHUNK, CHUNK)])

    k = pl.kernel(body, out_type=out_type, mesh=mesh, scratch_types=scratch,
                  compiler_params=_SC_PARAMS)
    return k(*idxs)


@functools.partial(jax.jit, static_argnames=("d", "signs"))
def _sc_scatter(vals, idxs, cnts, *, d, signs):
    ns = len(signs)
    dn = d * NP
    out_type = jax.ShapeDtypeStruct((NW, B, dn), F32)
    scratch = (
        [pltpu.VMEM((dn,), F32)]
        + [pltpu.VMEM((CHUNK,), jnp.int32) for _ in range(2 * ns)]
        + [pltpu.VMEM((d, CHUNK), F32) for _ in range(ns)]
    )
    mesh = plsc.VectorSubcoreMesh(core_axis_name="c", subcore_axis_name="s")

    def body(*refs):
        val_h = refs[0:ns]
        idx_h = refs[ns:2 * ns]
        cnt_h = refs[2 * ns:3 * ns]
        out_h = refs[3 * ns]
        sc = refs[3 * ns + 1:]
        acc_v = sc[0]
        idx_v = sc[1:1 + ns]
        cnt_v = sc[1 + ns:1 + 2 * ns]
        buf_v = sc[1 + 2 * ns:1 + 3 * ns]

        wid = _worker_id()
        base = wid * EPW
        zeros = jnp.zeros((LANE,), F32)
        for b in range(B):
            def zbody(i, _):
                for u in range(8):
                    acc_v[pl.ds(i * 8 * LANE + u * LANE, LANE)] = zeros
                return 0
            lax.fori_loop(0, dn // (8 * LANE), zbody, 0)
            for ck in range(EPW // CHUNK):
                for j in range(ns):
                    pltpu.sync_copy(
                        val_h[j].at[b, :, pl.ds(base + ck * CHUNK, CHUNK)],
                        buf_v[j])
                    pltpu.sync_copy(
                        idx_h[j].at[pl.ds(base + ck * CHUNK, CHUNK)],
                        idx_v[j])
                    pltpu.sync_copy(
                        cnt_h[j].at[pl.ds(base + ck * CHUNK, CHUNK)],
                        cnt_v[j])
                def inner(i, _, ck=ck):
                    for j in range(ns):
                        iv = idx_v[j][pl.ds(i * LANE, LANE)]
                        cnt = cnt_v[j][pl.ds(i * LANE, LANE)]
                        vs = []
                        for dd in range(d):
                            v = buf_v[j][dd, pl.ds(i * LANE, LANE)]
                            if signs[j] < 0:
                                v = -v
                            vs.append(v)
                        mk1 = cnt == 1
                        for dd in range(d):
                            plsc.addupdate_scatter(
                                acc_v, [iv + dd * NP], vs[dd], mask=mk1)
                        dups = plsc.all_reduce_population_count(cnt > 1)

                        @pl.when(dups[0] > 0)
                        def _(iv=iv, cnt=cnt, vs=vs):
                            rounds = lax.reduce_max(cnt, axes=(0,))

                            def sbody(rr, _, iv=iv, cnt=cnt, vs=vs):
                                mk = cnt == rr
                                for dd in range(d):
                                    plsc.addupdate_scatter(
                                        acc_v, [iv + dd * NP], vs[dd],
                                        mask=mk)
                                return 0
                            lax.fori_loop(2, rounds + 1, sbody, 0)
                    return 0
                lax.fori_loop(0, CHUNK // LANE, inner, 0)
            pltpu.sync_copy(acc_v, out_h.at[wid, b])

    k = pl.kernel(body, out_type=out_type, mesh=mesh, scratch_types=scratch,
                  compiler_params=_SC_PARAMS)
    return k(*vals, *idxs, *cnts)


def _rb(v):
    return v.astype(jnp.bfloat16).astype(F32)


def _mlp_fwd(xs, w1, b1, w2, b2, din, dhid, dout):
    hs = []
    for j in range(dhid):
        if din == 1:
            acc = xs[0] * w1[0, j]
        else:
            acc = _rb(xs[0]) * _rb(w1[0, j])
            for dd in range(1, din):
                acc = acc + _rb(xs[dd]) * _rb(w1[dd, j])
        hs.append(jnp.tanh(acc + b1[j]))
    outs = []
    for o in range(dout):
        acc = _rb(hs[0]) * _rb(w2[0, o])
        for j in range(1, dhid):
            acc = acc + _rb(hs[j]) * _rb(w2[j, o])
        outs.append(acc + b2[o])
    return outs, hs


def _mlp_bwd(douts, hs, w1, w2, din, dhid, dout):
    dxs = [None] * din
    for j in range(dhid):
        if dout == 1:
            acc = douts[0] * w2[j, 0]
        else:
            acc = _rb(douts[0]) * _rb(w2[j, 0])
            for o in range(1, dout):
                acc = acc + _rb(douts[o]) * _rb(w2[j, o])
        dz = acc * (1.0 - hs[j] * hs[j])
        dzr = _rb(dz)
        for dd in range(din):
            t = dzr * _rb(w1[dd, j])
            dxs[dd] = t if dxs[dd] is None else dxs[dd] + t
    return dxs


def _smem_spec(shape):
    return pl.BlockSpec(shape, lambda *_: (0,) * len(shape),
                        memory_space=pltpu.SMEM)


def _edge_spec(d):
    return pl.BlockSpec((B, d, EB), lambda i: (0, 0, i))


def _wargs(p):
    return (p["W1"], p["b1"], p["W2"], p["b2"])


def _wspecs(p):
    return [_smem_spec(p["W1"].shape), _smem_spec(p["b1"].shape),
            _smem_spec(p["W2"].shape), _smem_spec(p["b2"].shape)]


def _edge_out(d):
    return jax.ShapeDtypeStruct((B, d, EP), F32)


_GRID = EP // EB


def _t1(e, p_f1, p_f2):
    def kern(e_ref, w11, b11, w21, b21, w12, b12, w22, b22,
             ee_ref, hf1_ref, nu_ref, hf2_ref):
        ev = e_ref[:, 0, :]
        ee, hf1 = _mlp_fwd([ev], w11, b11, w21, b21, 1, 5, 5)
        nu, hf2 = _mlp_fwd(ee, w12, b12, w22, b22, 5, 5, 5)
        for dd in range(5):
            ee_ref[:, dd, :] = ee[dd]
            hf1_ref[:, dd, :] = hf1[dd]
            nu_ref[:, dd, :] = nu[dd]
            hf2_ref[:, dd, :] = hf2[dd]

    return pl.pallas_call(
        kern,
        grid=(_GRID,),
        in_specs=[_edge_spec(1)] + _wspecs(p_f1) + _wspecs(p_f2),
        out_specs=[_edge_spec(5)] * 4,
        out_shape=[_edge_out(5)] * 4,
    )(e, *_wargs(p_f1), *_wargs(p_f2))


def _t2(edge_embed, nes, ner, p_f3, p_f4, p_g2):
    def kern(ee_ref, nes_ref, ner_ref,
             w13, b13, w23, b23, w14, b14, w24, b24, w1g, b1g, w2g, b2g,
             v2_ref, h3_ref, h4_ref, hg2_ref):
        m = [nes_ref[:, dd, :] * ner_ref[:, dd, :] for dd in range(5)]
        n11, h3 = _mlp_fwd(m, w13, b13, w23, b23, 5, 5, 5)
        n12, h4 = _mlp_fwd(m, w14, b14, w24, b24, 5, 5, 5)
        ee = [((ee_ref[:, dd, :] + n11[dd]) + (ee_ref[:, dd, :] + n12[dd]))
              * 0.5 for dd in range(5)]
        v2, hg2 = _mlp_fwd(ee, w1g, b1g, w2g, b2g, 5, 5, 1)
        v2_ref[:, 0, :] = v2[0]
        for dd in range(5):
            h3_ref[:, dd, :] = h3[dd]
            h4_ref[:, dd, :] = h4[dd]
            hg2_ref[:, dd, :] = hg2[dd]

    return pl.pallas_call(
        kern,
        grid=(_GRID,),
        in_specs=[_edge_spec(5)] * 3 + _wspecs(p_f3) + _wspecs(p_f4)
        + _wspecs(p_g2),
        out_specs=[_edge_spec(1)] + [_edge_spec(5)] * 3,
        out_shape=[_edge_out(1)] + [_edge_out(5)] * 3,
    )(edge_embed, nes, ner, *_wargs(p_f3), *_wargs(p_f4), *_wargs(p_g2))


def _t4(dvvs, hg2, h3, h4, nes, ner, p_f3, p_f4, p_g2):
    def kern(dvvs_ref, hg2_ref, h3_ref, h4_ref, nes_ref, ner_ref,
             w13, b13, w23, b23, w14, b14, w24, b24, w1g, b1g, w2g, b2g,
             dee_ref, dnes_ref, dner_ref):
        dv2 = dvvs_ref[:, 0, :]
        hg2 = [hg2_ref[:, dd, :] for dd in range(5)]
        dee = _mlp_bwd([dv2], hg2, w1g, w2g, 5, 5, 1)
        deeh = [0.5 * x for x in dee]
        h3 = [h3_ref[:, dd, :] for dd in range(5)]
        h4 = [h4_ref[:, dd, :] for dd in range(5)]
        dma = _mlp_bwd(deeh, h3, w13, w23, 5, 5, 5)
        dmb = _mlp_bwd(deeh, h4, w14, w24, 5, 5, 5)
        for dd in range(5):
            dm = dma[dd] + dmb[dd]
            dee_ref[:, dd, :] = dee[dd]
            dnes_ref[:, dd, :] = dm * ner_ref[:, dd, :]
            dner_ref[:, dd, :] = dm * nes_ref[:, dd, :]

    return pl.pallas_call(
        kern,
        grid=(_GRID,),
        in_specs=[_edge_spec(1)] + [_edge_spec(5)] * 5 + _wspecs(p_f3)
        + _wspecs(p_f4) + _wspecs(p_g2),
        out_specs=[_edge_spec(5)] * 3,
        out_shape=[_edge_out(5)] * 3,
    )(dvvs, hg2, h3, h4, nes, ner, *_wargs(p_f3), *_wargs(p_f4),
      *_wargs(p_g2))


def _t5(dnus, hf2, dee, hf1, p_f1, p_f2):
    def kern(dnus_ref, hf2_ref, dee_ref, hf1_ref,
             w11, b11, w21, b21, w12, b12, w22, b22, de_ref):
        dnu = [dnus_ref[:, dd, :] for dd in range(5)]
        hf2 = [hf2_ref[:, dd, :] for dd in range(5)]
        dee2 = _mlp_bwd(dnu, hf2, w12, w22, 5, 5, 5)
        deet = [dee_ref[:, dd, :] + dee2[dd] for dd in range(5)]
        hf1 = [hf1_ref[:, dd, :] for dd in range(5)]
        de = _mlp_bwd(deet, hf1, w11, w21, 1, 5, 5)
        de_ref[:, 0, :] = de[0]

    return pl.pallas_call(
        kern,
        grid=(_GRID,),
        in_specs=[_edge_spec(5)] * 4 + _wspecs(p_f1) + _wspecs(p_f2),
        out_specs=_edge_spec(1),
        out_shape=_edge_out(1),
    )(dnus, hf2, dee, hf1, *_wargs(p_f1), *_wargs(p_f2))


def _fold(parts, extra=None, neg=False):
    nw, b_, w = parts.shape
    wb = 2560
    grid = w // wb

    def kern(*refs):
        if extra is None:
            p_ref, o_ref = refs
            s = jnp.sum(p_ref[...], axis=0)
        else:
            p_ref, e_ref, o_ref = refs
            s = jnp.sum(p_ref[...], axis=0) + e_ref[...]
        o_ref[...] = -s if neg else s

    in_specs = [pl.BlockSpec((nw, B, wb), lambda i: (0, 0, i))]
    args = [parts]
    if extra is not None:
        in_specs.append(pl.BlockSpec((B, wb), lambda i: (0, i)))
        args.append(extra)
    return pl.pallas_call(
        kern,
        grid=(grid,),
        in_specs=in_specs,
        out_specs=pl.BlockSpec((B, wb), lambda i: (0, i)),
        out_shape=jax.ShapeDtypeStruct((B, w), F32),
    )(*args)


def _t3(qpad, ppad, ne, vv_in, params):
    p_v1, p_v3, p_v, p_t = (params["V1"], params["V3"], params["V"],
                            params["T"])
    w1n2 = jnp.pad(params["net2"]["W1"], ((0, NP - N_NODE), (0, 0)))
    b1n2 = params["net2"]["b1"].reshape(1, -1)
    w2n2 = params["net2"]["W2"].reshape(1, -1)
    h2 = w1n2.shape[1]

    def kern(q_ref, p_ref, ne_ref, vv_ref, w1n2_ref, b1n2_ref, w2n2_ref,
             w1v1, b1v1, w2v1, b2v1, w1v3, b1v3, w2v3, b2v3,
             w1v, b1v, w2v, b2v, w1t, b1t, w2t, b2t,
             dvv_ref, dne_ref, dqn_ref, dt_ref):
        q = q_ref[...]
        vv = vv_ref[...]
        ne = [ne_ref[:, dd, :] for dd in range(5)]
        v1, hg1 = _mlp_fwd(ne, w1v1, b1v1, w2v1, b2v1, 5, 5, 1)
        v3, hg3 = _mlp_fwd([q], w1v3, b1v3, w2v3, b2v3, 1, 5, 1)
        u = [v1[0], vv, v3[0]]
        vth, hgv = _mlp_fwd(u, w1v, b1v, w2v, b2v, 3, 5, 1)
        h11 = vth[0]
        z = jnp.dot(h11, w1n2_ref[...],
                    preferred_element_type=F32) + b1n2_ref[...]
        a = jnp.tanh(z)
        g = (1.0 - a * a) * w2n2_ref[...]
        dh11 = lax.dot_general(g, w1n2_ref[...],
                               (((1,), (1,)), ((), ())),
                               preferred_element_type=F32)
        du = _mlp_bwd([dh11], hgv, w1v, w2v, 3, 5, 1)
        dvv_ref[...] = du[1]
        dqn_ref[...] = _mlp_bwd([du[2]], hg3, w1v3, w2v3, 1, 5, 1)[0]
        dne = _mlp_bwd([du[0]], hg1, w1v1, w2v1, 5, 5, 1)
        for dd in range(5):
            dne_ref[:, dd, :] = dne[dd]
        pv = p_ref[...]
        dt = None
        for j in range(10):
            ht = jnp.tanh(pv * w1t[0, j] + b1t[j])
            dz = w2t[j, 0] * (1.0 - ht * ht)
            t = _rb(dz) * _rb(w1t[0, j])
            dt = t if dt is None else dt + t
        dt_ref[...] = dt

    vm = lambda shape: pl.BlockSpec(shape, lambda: (0,) * len(shape))
    return pl.pallas_call(
        kern,
        in_specs=[vm((B, NP)), vm((B, NP)), vm((B, 5, NP)),
                  vm((B, NP)), vm((NP, h2)), vm((1, h2)), vm((1, h2))]
        + _wspecs(p_v1) + _wspecs(p_v3) + _wspecs(p_v) + _wspecs(p_t),
        out_specs=[vm((B, NP)), vm((B, 5, NP)), vm((B, NP)), vm((B, NP))],
        out_shape=[jax.ShapeDtypeStruct((B, NP), F32),
                   jax.ShapeDtypeStruct((B, 5, NP), F32),
                   jax.ShapeDtypeStruct((B, NP), F32),
                   jax.ShapeDtypeStruct((B, NP), F32)],
    )(qpad, ppad, ne, vv_in, w1n2, b1n2, w2n2,
      *_wargs(p_v1), *_wargs(p_v3), *_wargs(p_v), *_wargs(p_t))


def kernel(x, edge_index, params):
    n = N_NODE
    q = x[:, 0:n]
    p = x[:, n:2 * n]
    s = jnp.pad(edge_index[0], (0, EP - N_EDGE), constant_values=DUMP)
    r = jnp.pad(edge_index[1], (0, EP - N_EDGE), constant_values=DUMP)
    qpad = jnp.pad(q, ((0, 0), (0, NP - n)))
    ppad = jnp.pad(p, ((0, 0), (0, NP - n)))
    cnt_s, cnt_r = _sc_count((s, r))

    e = _sc_gather(qpad, (s, r), d=1, nidx=2, diff=True)[0]
    edge_embed, hf1, nu, hf2 = _t1(e, params["edge_init"],
                                   params["node_update"])
    nep = _sc_scatter((nu,), (s,), (cnt_s,), d=5, signs=(1,))
    ne_flat = _fold(nep)
    ne = ne_flat.reshape(B, 5, NP)

    nes, ner = _sc_gather(ne_flat, (s, r), d=5, nidx=2, diff=False)
    v2, h3, h4, hg2 = _t2(edge_embed, nes, ner, params["edge_up1"],
                          params["edge_up2"], params["V2"])
    vvp = _sc_scatter((v2,), (s,), (cnt_s,), d=1, signs=(1,))
    vv = _fold(vvp)

    dvv, dne_node, dqn, dt = _t3(qpad, ppad, ne, vv, params)

    dvvs = _sc_gather(dvv, (s,), d=1, nidx=1, diff=False)[0]
    dee, dnes, dner = _t4(dvvs, hg2, h3, h4, nes, ner,
                          params["edge_up1"], params["edge_up2"],
                          params["V2"])
    dnep = _sc_scatter((dnes, dner), (s, r), (cnt_s, cnt_r), d=5, signs=(1, 1))
    dne_flat = _fold(dnep, extra=dne_node.reshape(B, 5 * NP))

    dnus = _sc_gather(dne_flat, (s,), d=5, nidx=1, diff=False)[0]
    de = _t5(dnus, hf2, dee, hf1, params["edge_init"],
             params["node_update"])
    dqp = _sc_scatter((de, de), (s, r), (cnt_s, cnt_r), d=1, signs=(1, -1))
    mdq = _fold(dqp, extra=dqn, neg=True)

    return jnp.concatenate([dt[:, :n], mdq[:, :n]], axis=1)
